# baseline (speedup 1.0000x reference)
import jax
import jax.numpy as jnp
from jax.experimental import pallas as pl
from jax.experimental.pallas import tpu as pltpu

_GRID = 26


def _copy_body(x_ref, o_ref):
    o_ref[...] = x_ref[...]


def kernel(features, is_start):
    del is_start
    xt = jnp.transpose(features, (2, 3, 0, 1))
    h, w, b, c = xt.shape
    step = h // _GRID
    out = pl.pallas_call(
        _copy_body,
        grid=(_GRID,),
        in_specs=[pl.BlockSpec((step, w, b, c), lambda i: (i, 0, 0, 0))],
        out_specs=pl.BlockSpec((step, w, b, c), lambda i: (i, 0, 0, 0)),
        out_shape=jax.ShapeDtypeStruct(xt.shape, xt.dtype),
        compiler_params=pltpu.CompilerParams(
            dimension_semantics=("arbitrary",),
        ),
    )(xt)
    return jnp.transpose(out, (2, 3, 0, 1))


# speedup vs baseline: 1.6470x; 1.6470x over previous
import jax
import jax.numpy as jnp
from jax.experimental import pallas as pl
from jax.experimental.pallas import tpu as pltpu

_GRID = 4


def _copy_body(x_ref, o_ref):
    o_ref[...] = x_ref[...]


def kernel(features, is_start):
    del is_start
    xt = jnp.transpose(features, (2, 3, 0, 1))
    h, w, b, c = xt.shape
    step = h // _GRID
    out = pl.pallas_call(
        _copy_body,
        grid=(_GRID,),
        in_specs=[pl.BlockSpec((step, w, b, c), lambda i: (i, 0, 0, 0))],
        out_specs=pl.BlockSpec((step, w, b, c), lambda i: (i, 0, 0, 0)),
        out_shape=jax.ShapeDtypeStruct(xt.shape, xt.dtype),
        compiler_params=pltpu.CompilerParams(
            dimension_semantics=("arbitrary",),
        ),
    )(xt)
    return jnp.transpose(out, (2, 3, 0, 1))


# speedup vs baseline: 1.7867x; 1.0848x over previous
import jax
import jax.numpy as jnp
from jax.experimental import pallas as pl
from jax.experimental.pallas import tpu as pltpu

_GRID = 2


def _copy_body(x_ref, o_ref):
    o_ref[...] = x_ref[...]


def kernel(features, is_start):
    del is_start
    xt = jnp.transpose(features, (2, 3, 0, 1))
    h, w, b, c = xt.shape
    step = h // _GRID
    out = pl.pallas_call(
        _copy_body,
        grid=(_GRID,),
        in_specs=[pl.BlockSpec((step, w, b, c), lambda i: (i, 0, 0, 0))],
        out_specs=pl.BlockSpec((step, w, b, c), lambda i: (i, 0, 0, 0)),
        out_shape=jax.ShapeDtypeStruct(xt.shape, xt.dtype),
        compiler_params=pltpu.CompilerParams(
            dimension_semantics=("arbitrary",),
        ),
    )(xt)
    return jnp.transpose(out, (2, 3, 0, 1))
